# Initial kernel scaffold; baseline (speedup 1.0000x reference)
import jax
import jax.numpy as jnp
from jax.experimental import pallas as pl


def kernel(input, means, weight_sum):
    raise NotImplementedError("write your pallas kernel here")


# speedup vs baseline: 3.0403x; 3.0403x over previous
import functools

import jax
import jax.numpy as jnp
from jax.experimental import pallas as pl
from jax.experimental.pallas import tpu as pltpu

_BLOCK = 2048


def _kmeans_body(x_ref, m_ref, ws_ref, out_ref, inertia_ref,
                 sums_ref, counts_ref, acc_ref):
    i = pl.program_id(0)
    nblk = pl.num_programs(0)

    x = x_ref[...]
    m = m_ref[0]

    @pl.when(i == 0)
    def _init():
        sums_ref[...] = jnp.zeros_like(sums_ref)
        counts_ref[...] = jnp.zeros_like(counts_ref)
        acc_ref[0, 0] = 0.0

    dots = jax.lax.dot_general(x, m, (((1,), (1,)), ((), ())),
                               preferred_element_type=jnp.float32)
    m2 = jnp.sum(m * m, axis=1)
    x2 = jnp.sum(x * x, axis=1)
    dist = (x2[:, None] + m2[None, :]) - 2.0 * dots

    bins = jnp.argmin(dist, axis=1)
    selected = jnp.min(dist, axis=1)
    acc_ref[0, 0] += jnp.sum(jnp.sqrt(selected))

    K = m.shape[0]
    onehot = (bins[:, None] == jax.lax.broadcasted_iota(jnp.int32, (1, K), 1)
              ).astype(jnp.float32)
    sums_ref[...] += jax.lax.dot_general(
        onehot, x, (((0,), (0,)), ((), ())),
        preferred_element_type=jnp.float32)
    counts_ref[...] += jnp.sum(onehot, axis=0, keepdims=True)

    @pl.when(i == nblk - 1)
    def _finalize():
        ws = ws_ref[0]
        counts = counts_ref[0]
        total = ws + counts
        alpha = 1.0 / jnp.where(total == 0.0, 1.0, total)
        iszero = (counts == 0.0).astype(jnp.float32)
        nm = (sums_ref[...] + m * ws[:, None]) * alpha[:, None]
        out_ref[0] = m * iszero[:, None] + nm * (1.0 - iszero[:, None])
        inertia_ref[0, 0] = acc_ref[0, 0]


@jax.jit
def kernel(input, means, weight_sum):
    N, D = input.shape
    G, K, _ = means.shape
    grid = N // _BLOCK

    new_means, inertia = pl.pallas_call(
        _kmeans_body,
        grid=(grid,),
        in_specs=[
            pl.BlockSpec((_BLOCK, D), lambda i: (i, 0)),
            pl.BlockSpec((1, K, D), lambda i: (0, 0, 0)),
            pl.BlockSpec((1, K), lambda i: (0, 0)),
        ],
        out_specs=[
            pl.BlockSpec((1, K, D), lambda i: (0, 0, 0)),
            pl.BlockSpec((1, 1), lambda i: (0, 0), memory_space=pltpu.SMEM),
        ],
        out_shape=[
            jax.ShapeDtypeStruct((1, K, D), jnp.float32),
            jax.ShapeDtypeStruct((1, 1), jnp.float32),
        ],
        scratch_shapes=[
            pltpu.VMEM((K, D), jnp.float32),
            pltpu.VMEM((1, K), jnp.float32),
            pltpu.SMEM((1, 1), jnp.float32),
        ],
    )(input, means, weight_sum)
    return new_means, inertia[0, 0]


# speedup vs baseline: 4.0228x; 1.3232x over previous
import functools

import jax
import jax.numpy as jnp
from jax.experimental import pallas as pl
from jax.experimental.pallas import tpu as pltpu

_BLOCK = 2048


def _kmeans_body(x_ref, m_ref, ws_ref, out_ref, inertia_ref,
                 sums_ref, counts_ref, acc_ref, mneg2_ref, m2_ref):
    i = pl.program_id(0)
    nblk = pl.num_programs(0)

    x = x_ref[...]

    @pl.when(i == 0)
    def _init():
        m0 = m_ref[0]
        sums_ref[...] = jnp.zeros_like(sums_ref)
        counts_ref[...] = jnp.zeros_like(counts_ref)
        acc_ref[0, 0] = 0.0
        mneg2_ref[...] = -2.0 * m0
        m2_ref[...] = jnp.sum(m0 * m0, axis=1)[None, :]

    dots = jax.lax.dot_general(x, mneg2_ref[...], (((1,), (1,)), ((), ())),
                               preferred_element_type=jnp.float32)
    x2 = jnp.sum(x * x, axis=1)
    dist = (x2[:, None] + m2_ref[...]) + dots

    bins = jnp.argmin(dist, axis=1)
    mn = jnp.min(dist, axis=1)
    acc_ref[0, 0] += jnp.sum(jnp.sqrt(mn))

    K = mneg2_ref.shape[0]
    B = x.shape[0]
    onehot = (bins[:, None] == jax.lax.broadcasted_iota(jnp.int32, (1, K), 1)
              ).astype(jnp.float32)
    sums_ref[...] += jax.lax.dot_general(
        onehot, x, (((0,), (0,)), ((), ())),
        preferred_element_type=jnp.float32)
    counts_ref[...] += jax.lax.dot_general(
        jnp.ones((1, B), jnp.float32), onehot, (((1,), (0,)), ((), ())),
        preferred_element_type=jnp.float32)

    @pl.when(i == nblk - 1)
    def _finalize():
        m = m_ref[0]
        ws = ws_ref[0]
        counts = counts_ref[0]
        total = ws + counts
        alpha = 1.0 / jnp.where(total == 0.0, 1.0, total)
        iszero = (counts == 0.0).astype(jnp.float32)
        nm = (sums_ref[...] + m * ws[:, None]) * alpha[:, None]
        out_ref[0] = m * iszero[:, None] + nm * (1.0 - iszero[:, None])
        inertia_ref[0, 0] = acc_ref[0, 0]


@jax.jit
def kernel(input, means, weight_sum):
    N, D = input.shape
    G, K, _ = means.shape
    grid = N // _BLOCK

    new_means, inertia = pl.pallas_call(
        _kmeans_body,
        grid=(grid,),
        in_specs=[
            pl.BlockSpec((_BLOCK, D), lambda i: (i, 0)),
            pl.BlockSpec((1, K, D), lambda i: (0, 0, 0)),
            pl.BlockSpec((1, K), lambda i: (0, 0)),
        ],
        out_specs=[
            pl.BlockSpec((1, K, D), lambda i: (0, 0, 0)),
            pl.BlockSpec((1, 1), lambda i: (0, 0), memory_space=pltpu.SMEM),
        ],
        out_shape=[
            jax.ShapeDtypeStruct((1, K, D), jnp.float32),
            jax.ShapeDtypeStruct((1, 1), jnp.float32),
        ],
        scratch_shapes=[
            pltpu.VMEM((K, D), jnp.float32),
            pltpu.VMEM((1, K), jnp.float32),
            pltpu.SMEM((1, 1), jnp.float32),
            pltpu.VMEM((K, D), jnp.float32),
            pltpu.VMEM((1, K), jnp.float32),
        ],
    )(input, means, weight_sum)
    return new_means, inertia[0, 0]
